# Initial kernel scaffold; baseline (speedup 1.0000x reference)
import jax
import jax.numpy as jnp
from jax.experimental import pallas as pl


def kernel(x, dw_w, pw_w, g1, b1, g2, b2):
    raise NotImplementedError("write your pallas kernel here")


# speedup vs baseline: 2.1534x; 2.1534x over previous
import functools

import jax
import jax.numpy as jnp
from jax.experimental import pallas as pl
from jax.experimental.pallas import tpu as pltpu

EPS = 1e-5
f32 = jnp.float32


def _shift_rows(p, d):
    if d == 0:
        return p
    if d == 1:
        return jnp.concatenate([p[1:, :], p[:1, :]], axis=0)
    return jnp.concatenate([p[-1:, :], p[:-1, :]], axis=0)


def _shift_lanes(p, d):
    if d == 0:
        return p
    if d == 1:
        return jnp.concatenate([p[:, 1:], p[:, :1]], axis=1)
    return jnp.concatenate([p[:, -1:], p[:, :-1]], axis=1)


def _build_planes(x_ref, nin, h, w):
    zc = jnp.zeros((h, 1), f32)
    zr = jnp.zeros((1, 2 * (w + 2)), f32)
    planes = []
    for c in range(nin):
        row = jnp.concatenate(
            [zc, x_ref[0, c], zc, zc, x_ref[1, c], zc], axis=1)
        planes.append(jnp.concatenate([zr, row, zr], axis=0))
    return planes


def _depthwise(planes, dwb_ref, kpl):
    zs = []
    for c, p in enumerate(planes):
        rows = [_shift_rows(p, dh) for dh in (-1, 0, 1)]
        accs = [None] * kpl
        for kh in range(3):
            for kw in range(3):
                tap = _shift_lanes(rows[kh], kw - 1)
                j = kh * 3 + kw
                for k in range(kpl):
                    m = c * kpl + k
                    t = tap * dwb_ref[m * 9 + j:m * 9 + j + 1, :]
                    accs[k] = t if accs[k] is None else accs[k] + t
        zs.extend(accs)
    return zs


def _interior_mask(hp, l):
    wp = l // 2
    r = jax.lax.broadcasted_iota(jnp.int32, (hp, l), 0)
    q = jax.lax.broadcasted_iota(jnp.int32, (hp, l), 1)
    q = jax.lax.rem(q, wp)
    ok = (r >= 1) & (r <= hp - 2) & (q >= 1) & (q <= wp - 2)
    return jnp.where(ok, 1.0, 0.0).astype(f32)


def _pointwise(acts, pwb_ref, nout):
    cmid = len(acts)
    outs = [None] * nout
    for m in range(cmid):
        for o in range(nout):
            t = acts[m] * pwb_ref[o * cmid + m:o * cmid + m + 1, :]
            outs[o] = t if outs[o] is None else outs[o] + t
    return outs


def _stats1_kernel(x_ref, dwb_ref, s_ref, ss_ref, *, nin, kpl, h, w):
    @pl.when(pl.program_id(0) == 0)
    def _():
        s_ref[...] = jnp.zeros_like(s_ref)
        ss_ref[...] = jnp.zeros_like(ss_ref)

    zs = _depthwise(_build_planes(x_ref, nin, h, w), dwb_ref, kpl)
    mask = _interior_mask(h + 2, 2 * (w + 2))
    srows, ssrows = [], []
    for z in zs:
        zm = z * mask
        srows.append(jnp.sum(zm, axis=0, keepdims=True))
        ssrows.append(jnp.sum(zm * zm, axis=0, keepdims=True))
    s_ref[...] += jnp.concatenate(srows, axis=0)
    ss_ref[...] += jnp.concatenate(ssrows, axis=0)


def _stats2_kernel(x_ref, dwb_ref, a1_ref, b1_ref, pwb_ref, s_ref, ss_ref,
                   *, nin, kpl, nout, h, w):
    @pl.when(pl.program_id(0) == 0)
    def _():
        s_ref[...] = jnp.zeros_like(s_ref)
        ss_ref[...] = jnp.zeros_like(ss_ref)

    zs = _depthwise(_build_planes(x_ref, nin, h, w), dwb_ref, kpl)
    mask = _interior_mask(h + 2, 2 * (w + 2))
    acts = [jnp.maximum(z * a1_ref[m:m + 1, :] + b1_ref[m:m + 1, :], 0.0)
            * mask for m, z in enumerate(zs)]
    pws = _pointwise(acts, pwb_ref, nout)
    srows, ssrows = [], []
    for pw in pws:
        srows.append(jnp.sum(pw, axis=0, keepdims=True))
        ssrows.append(jnp.sum(pw * pw, axis=0, keepdims=True))
    s_ref[...] += jnp.concatenate(srows, axis=0)
    ss_ref[...] += jnp.concatenate(ssrows, axis=0)


def _final_kernel(x_ref, dwb_ref, a1_ref, b1_ref, pwb_ref, a2_ref, b2_ref,
                  out_ref, *, nin, kpl, nout, h, w):
    zs = _depthwise(_build_planes(x_ref, nin, h, w), dwb_ref, kpl)
    mask = _interior_mask(h + 2, 2 * (w + 2))
    acts = [jnp.maximum(z * a1_ref[m:m + 1, :] + b1_ref[m:m + 1, :], 0.0)
            * mask for m, z in enumerate(zs)]
    pws = _pointwise(acts, pwb_ref, nout)
    wp = w + 2
    for o, pw in enumerate(pws):
        val = jnp.maximum(pw * a2_ref[o:o + 1, :] + b2_ref[o:o + 1, :], 0.0)
        out_ref[0, o] = val[:, :wp]
        out_ref[1, o] = val[:, wp:]


def kernel(x, dw_w, pw_w, g1, b1, g2, b2):
    N, NIN, H, W = x.shape
    CMID = dw_w.shape[0]
    NOUT = pw_w.shape[0]
    KPL = CMID // NIN
    Hp, Wp = H + 2, W + 2
    L = 2 * Wp
    N2 = N // 2

    dwb = jnp.broadcast_to(dw_w.astype(f32).reshape(CMID * 9, 1), (CMID * 9, L))
    pwm = pw_w.astype(f32)[:, :, 0, 0]
    pwb = jnp.broadcast_to(pwm.reshape(NOUT * CMID, 1), (NOUT * CMID, L))

    x_spec = pl.BlockSpec((2, NIN, H, W), lambda n: (n, 0, 0, 0))

    def cspec(shape):
        nd = len(shape)
        return pl.BlockSpec(shape, lambda n, nd=nd: (0,) * nd)

    cp = pltpu.CompilerParams(dimension_semantics=("arbitrary",),
                              vmem_limit_bytes=48 * 1024 * 1024)

    s1, ss1 = pl.pallas_call(
        functools.partial(_stats1_kernel, nin=NIN, kpl=KPL, h=H, w=W),
        out_shape=(jax.ShapeDtypeStruct((CMID, L), f32),
                   jax.ShapeDtypeStruct((CMID, L), f32)),
        grid=(N2,),
        in_specs=[x_spec, cspec((CMID * 9, L))],
        out_specs=(cspec((CMID, L)), cspec((CMID, L))),
        compiler_params=cp,
    )(x, dwb)

    cnt1 = float(N * H * W)
    mean1 = jnp.sum(s1, axis=1) / cnt1
    var1 = jnp.maximum(jnp.sum(ss1, axis=1) / cnt1 - mean1 * mean1, 0.0)
    a1 = g1.astype(f32) * jax.lax.rsqrt(var1 + EPS)
    b1v = b1.astype(f32) - mean1 * a1
    a1b = jnp.broadcast_to(a1.reshape(CMID, 1), (CMID, L))
    b1b = jnp.broadcast_to(b1v.reshape(CMID, 1), (CMID, L))

    s2, ss2 = pl.pallas_call(
        functools.partial(_stats2_kernel, nin=NIN, kpl=KPL, nout=NOUT,
                          h=H, w=W),
        out_shape=(jax.ShapeDtypeStruct((NOUT, L), f32),
                   jax.ShapeDtypeStruct((NOUT, L), f32)),
        grid=(N2,),
        in_specs=[x_spec, cspec((CMID * 9, L)), cspec((CMID, L)),
                  cspec((CMID, L)), cspec((NOUT * CMID, L))],
        out_specs=(cspec((NOUT, L)), cspec((NOUT, L))),
        compiler_params=cp,
    )(x, dwb, a1b, b1b, pwb)

    cnt2 = float(N * Hp * Wp)
    mean2 = jnp.sum(s2, axis=1) / cnt2
    var2 = jnp.maximum(jnp.sum(ss2, axis=1) / cnt2 - mean2 * mean2, 0.0)
    a2 = g2.astype(f32) * jax.lax.rsqrt(var2 + EPS)
    b2v = b2.astype(f32) - mean2 * a2
    a2b = jnp.broadcast_to(a2.reshape(NOUT, 1), (NOUT, L))
    b2b = jnp.broadcast_to(b2v.reshape(NOUT, 1), (NOUT, L))

    out = pl.pallas_call(
        functools.partial(_final_kernel, nin=NIN, kpl=KPL, nout=NOUT,
                          h=H, w=W),
        out_shape=jax.ShapeDtypeStruct((N, NOUT, Hp, Wp), f32),
        grid=(N2,),
        in_specs=[x_spec, cspec((CMID * 9, L)), cspec((CMID, L)),
                  cspec((CMID, L)), cspec((NOUT * CMID, L)),
                  cspec((NOUT, L)), cspec((NOUT, L))],
        out_specs=pl.BlockSpec((2, NOUT, Hp, Wp), lambda n: (n, 0, 0, 0)),
        compiler_params=cp,
    )(x, dwb, a1b, b1b, pwb, a2b, b2b)
    return out


# speedup vs baseline: 3.2183x; 1.4945x over previous
import functools

import jax
import jax.numpy as jnp
from jax.experimental import pallas as pl
from jax.experimental.pallas import tpu as pltpu

EPS = 1e-5
f32 = jnp.float32


def _shift_rows(p, d):
    if d == 0:
        return p
    if d == 1:
        return jnp.concatenate([p[1:, :], p[:1, :]], axis=0)
    return jnp.concatenate([p[-1:, :], p[:-1, :]], axis=0)


def _shift_lanes(p, d):
    if d == 0:
        return p
    if d == 1:
        return jnp.concatenate([p[:, 1:], p[:, :1]], axis=1)
    return jnp.concatenate([p[:, -1:], p[:, :-1]], axis=1)


def _build_planes(x_ref, b0, nin, h, w):
    zc = jnp.zeros((h, 1), f32)
    zr = jnp.zeros((1, 2 * (w + 2)), f32)
    planes = []
    for c in range(nin):
        row = jnp.concatenate(
            [zc, x_ref[b0, c], zc, zc, x_ref[b0 + 1, c], zc], axis=1)
        planes.append(jnp.concatenate([zr, row, zr], axis=0))
    return planes


def _depthwise(planes, dwb_ref, kpl):
    zs = []
    for c, p in enumerate(planes):
        rows = [_shift_rows(p, dh) for dh in (-1, 0, 1)]
        accs = [None] * kpl
        for kh in range(3):
            for kw in range(3):
                tap = _shift_lanes(rows[kh], kw - 1)
                j = kh * 3 + kw
                for k in range(kpl):
                    m = c * kpl + k
                    t = tap * dwb_ref[m * 9 + j:m * 9 + j + 1, :]
                    accs[k] = t if accs[k] is None else accs[k] + t
        zs.extend(accs)
    return zs


def _interior_mask(hp, l):
    wp = l // 2
    r = jax.lax.broadcasted_iota(jnp.int32, (hp, l), 0)
    q = jax.lax.broadcasted_iota(jnp.int32, (hp, l), 1)
    q = jax.lax.rem(q, wp)
    ok = (r >= 1) & (r <= hp - 2) & (q >= 1) & (q <= wp - 2)
    return jnp.where(ok, 1.0, 0.0).astype(f32)


def _pointwise(acts, pwb_ref, nout):
    cmid = len(acts)
    outs = [None] * nout
    for m in range(cmid):
        for o in range(nout):
            t = acts[m] * pwb_ref[o * cmid + m:o * cmid + m + 1, :]
            outs[o] = t if outs[o] is None else outs[o] + t
    return outs


def _stats1_kernel(x_ref, dwb_ref, s_ref, ss_ref, *, nb, nin, kpl, h, w):
    @pl.when(pl.program_id(0) == 0)
    def _():
        s_ref[...] = jnp.zeros_like(s_ref)
        ss_ref[...] = jnp.zeros_like(ss_ref)

    mask = _interior_mask(h + 2, 2 * (w + 2))
    srows, ssrows = None, None
    for p in range(nb // 2):
        zs = _depthwise(_build_planes(x_ref, 2 * p, nin, h, w), dwb_ref, kpl)
        sr = []
        sq = []
        for z in zs:
            zm = z * mask
            sr.append(jnp.sum(zm, axis=0, keepdims=True))
            sq.append(jnp.sum(zm * zm, axis=0, keepdims=True))
        srows = sr if srows is None else [a + b for a, b in zip(srows, sr)]
        ssrows = sq if ssrows is None else [a + b for a, b in zip(ssrows, sq)]
    s_ref[...] += jnp.concatenate(srows, axis=0)
    ss_ref[...] += jnp.concatenate(ssrows, axis=0)


def _stats2_kernel(x_ref, dwb_ref, a1_ref, b1_ref, pwb_ref, s_ref, ss_ref,
                   *, nb, nin, kpl, nout, h, w):
    @pl.when(pl.program_id(0) == 0)
    def _():
        s_ref[...] = jnp.zeros_like(s_ref)
        ss_ref[...] = jnp.zeros_like(ss_ref)

    mask = _interior_mask(h + 2, 2 * (w + 2))
    srows, ssrows = None, None
    for p in range(nb // 2):
        zs = _depthwise(_build_planes(x_ref, 2 * p, nin, h, w), dwb_ref, kpl)
        acts = [jnp.maximum(z * a1_ref[m:m + 1, :] + b1_ref[m:m + 1, :], 0.0)
                * mask for m, z in enumerate(zs)]
        pws = _pointwise(acts, pwb_ref, nout)
        sr = []
        sq = []
        for pw in pws:
            sr.append(jnp.sum(pw, axis=0, keepdims=True))
            sq.append(jnp.sum(pw * pw, axis=0, keepdims=True))
        srows = sr if srows is None else [a + b for a, b in zip(srows, sr)]
        ssrows = sq if ssrows is None else [a + b for a, b in zip(ssrows, sq)]
    s_ref[...] += jnp.concatenate(srows, axis=0)
    ss_ref[...] += jnp.concatenate(ssrows, axis=0)


def _final_kernel(x_ref, dwb_ref, a1_ref, b1_ref, pwb_ref, a2_ref, b2_ref,
                  out_ref, *, nb, nin, kpl, nout, h, w):
    mask = _interior_mask(h + 2, 2 * (w + 2))
    wp = w + 2
    for p in range(nb // 2):
        zs = _depthwise(_build_planes(x_ref, 2 * p, nin, h, w), dwb_ref, kpl)
        acts = [jnp.maximum(z * a1_ref[m:m + 1, :] + b1_ref[m:m + 1, :], 0.0)
                * mask for m, z in enumerate(zs)]
        pws = _pointwise(acts, pwb_ref, nout)
        for o, pw in enumerate(pws):
            val = jnp.maximum(pw * a2_ref[o:o + 1, :] + b2_ref[o:o + 1, :],
                              0.0)
            out_ref[2 * p, o] = val[:, :wp]
            out_ref[2 * p + 1, o] = val[:, wp:]


def kernel(x, dw_w, pw_w, g1, b1, g2, b2):
    N, NIN, H, W = x.shape
    CMID = dw_w.shape[0]
    NOUT = pw_w.shape[0]
    KPL = CMID // NIN
    Hp, Wp = H + 2, W + 2
    L = 2 * Wp
    NB = 8 if N % 8 == 0 else 2
    NBLK = N // NB

    dwb = jnp.broadcast_to(dw_w.astype(f32).reshape(CMID * 9, 1), (CMID * 9, L))
    pwm = pw_w.astype(f32)[:, :, 0, 0]
    pwb = jnp.broadcast_to(pwm.reshape(NOUT * CMID, 1), (NOUT * CMID, L))

    x_spec = pl.BlockSpec((NB, NIN, H, W), lambda n: (n, 0, 0, 0))

    def cspec(shape):
        nd = len(shape)
        return pl.BlockSpec(shape, lambda n, nd=nd: (0,) * nd)

    cp = pltpu.CompilerParams(dimension_semantics=("arbitrary",),
                              vmem_limit_bytes=48 * 1024 * 1024)

    s1, ss1 = pl.pallas_call(
        functools.partial(_stats1_kernel, nb=NB, nin=NIN, kpl=KPL, h=H, w=W),
        out_shape=(jax.ShapeDtypeStruct((CMID, L), f32),
                   jax.ShapeDtypeStruct((CMID, L), f32)),
        grid=(NBLK,),
        in_specs=[x_spec, cspec((CMID * 9, L))],
        out_specs=(cspec((CMID, L)), cspec((CMID, L))),
        compiler_params=cp,
    )(x, dwb)

    cnt1 = float(N * H * W)
    mean1 = jnp.sum(s1, axis=1) / cnt1
    var1 = jnp.maximum(jnp.sum(ss1, axis=1) / cnt1 - mean1 * mean1, 0.0)
    a1 = g1.astype(f32) * jax.lax.rsqrt(var1 + EPS)
    b1v = b1.astype(f32) - mean1 * a1
    a1b = jnp.broadcast_to(a1.reshape(CMID, 1), (CMID, L))
    b1b = jnp.broadcast_to(b1v.reshape(CMID, 1), (CMID, L))

    s2, ss2 = pl.pallas_call(
        functools.partial(_stats2_kernel, nb=NB, nin=NIN, kpl=KPL,
                          nout=NOUT, h=H, w=W),
        out_shape=(jax.ShapeDtypeStruct((NOUT, L), f32),
                   jax.ShapeDtypeStruct((NOUT, L), f32)),
        grid=(NBLK,),
        in_specs=[x_spec, cspec((CMID * 9, L)), cspec((CMID, L)),
                  cspec((CMID, L)), cspec((NOUT * CMID, L))],
        out_specs=(cspec((NOUT, L)), cspec((NOUT, L))),
        compiler_params=cp,
    )(x, dwb, a1b, b1b, pwb)

    cnt2 = float(N * Hp * Wp)
    mean2 = jnp.sum(s2, axis=1) / cnt2
    var2 = jnp.maximum(jnp.sum(ss2, axis=1) / cnt2 - mean2 * mean2, 0.0)
    a2 = g2.astype(f32) * jax.lax.rsqrt(var2 + EPS)
    b2v = b2.astype(f32) - mean2 * a2
    a2b = jnp.broadcast_to(a2.reshape(NOUT, 1), (NOUT, L))
    b2b = jnp.broadcast_to(b2v.reshape(NOUT, 1), (NOUT, L))

    out = pl.pallas_call(
        functools.partial(_final_kernel, nb=NB, nin=NIN, kpl=KPL,
                          nout=NOUT, h=H, w=W),
        out_shape=jax.ShapeDtypeStruct((N, NOUT, Hp, Wp), f32),
        grid=(NBLK,),
        in_specs=[x_spec, cspec((CMID * 9, L)), cspec((CMID, L)),
                  cspec((CMID, L)), cspec((NOUT * CMID, L)),
                  cspec((NOUT, L)), cspec((NOUT, L))],
        out_specs=pl.BlockSpec((NB, NOUT, Hp, Wp), lambda n: (n, 0, 0, 0)),
        compiler_params=cp,
    )(x, dwb, a1b, b1b, pwb, a2b, b2b)
    return out


# speedup vs baseline: 4.0535x; 1.2595x over previous
import functools

import jax
import jax.numpy as jnp
from jax.experimental import pallas as pl
from jax.experimental.pallas import tpu as pltpu

EPS = 1e-5
f32 = jnp.float32


def _shift_rows(p, d):
    if d == 0:
        return p
    if d == 1:
        return jnp.concatenate([p[1:, :], p[:1, :]], axis=0)
    return jnp.concatenate([p[-1:, :], p[:-1, :]], axis=0)


def _shift_lanes(p, d):
    if d == 0:
        return p
    if d == 1:
        return jnp.concatenate([p[:, 1:], p[:, :1]], axis=1)
    return jnp.concatenate([p[:, -1:], p[:, :-1]], axis=1)


def _build_planes(x_ref, b0, nin, h, w):
    zc = jnp.zeros((h, 1), f32)
    zr = jnp.zeros((1, 2 * (w + 2)), f32)
    planes = []
    for c in range(nin):
        row = jnp.concatenate(
            [zc, x_ref[b0, c], zc, zc, x_ref[b0 + 1, c], zc], axis=1)
        planes.append(jnp.concatenate([zr, row, zr], axis=0))
    return planes


def _depthwise(planes, dwb_ref, kpl):
    zs = []
    for c, p in enumerate(planes):
        rows = [_shift_rows(p, dh) for dh in (-1, 0, 1)]
        accs = [None] * kpl
        for kh in range(3):
            for kw in range(3):
                tap = _shift_lanes(rows[kh], kw - 1)
                j = kh * 3 + kw
                for k in range(kpl):
                    m = c * kpl + k
                    t = tap * dwb_ref[m * 9 + j:m * 9 + j + 1, :]
                    accs[k] = t if accs[k] is None else accs[k] + t
        zs.extend(accs)
    return zs


def _interior_mask(hp, l):
    wp = l // 2
    r = jax.lax.broadcasted_iota(jnp.int32, (hp, l), 0)
    q = jax.lax.broadcasted_iota(jnp.int32, (hp, l), 1)
    q = jax.lax.rem(q, wp)
    ok = (r >= 1) & (r <= hp - 2) & (q >= 1) & (q <= wp - 2)
    return jnp.where(ok, 1.0, 0.0).astype(f32)


def _pointwise(acts, pwb_ref, nout):
    cmid = len(acts)
    outs = [None] * nout
    for m in range(cmid):
        for o in range(nout):
            t = acts[m] * pwb_ref[o * cmid + m:o * cmid + m + 1, :]
            outs[o] = t if outs[o] is None else outs[o] + t
    return outs


def _stats1_kernel(x_ref, dwb_ref, s_ref, ss_ref, z_ref,
                   *, nb, nin, kpl, h, w):
    @pl.when(pl.program_id(1) == 0)
    def _():
        s_ref[...] = jnp.zeros_like(s_ref)
        ss_ref[...] = jnp.zeros_like(ss_ref)

    mask = _interior_mask(h + 2, 2 * (w + 2))
    srows, ssrows = None, None
    for p in range(nb // 2):
        zs = _depthwise(_build_planes(x_ref, 2 * p, nin, h, w), dwb_ref, kpl)
        sr = []
        sq = []
        for m, z in enumerate(zs):
            z_ref[p, m] = z.astype(jnp.bfloat16)
            zm = z * mask
            sr.append(jnp.sum(zm, axis=0, keepdims=True))
            sq.append(jnp.sum(zm * zm, axis=0, keepdims=True))
        srows = sr if srows is None else [a + b for a, b in zip(srows, sr)]
        ssrows = sq if ssrows is None else [a + b for a, b in zip(ssrows, sq)]
    s_ref[0] += jnp.concatenate(srows, axis=0)
    ss_ref[0] += jnp.concatenate(ssrows, axis=0)


def _stats2_kernel(z_ref, a1_ref, b1_ref, pwb_ref, s_ref, ss_ref,
                   *, nb, cmid, nout, h, w):
    @pl.when(pl.program_id(1) == 0)
    def _():
        s_ref[...] = jnp.zeros_like(s_ref)
        ss_ref[...] = jnp.zeros_like(ss_ref)

    mask = _interior_mask(h + 2, 2 * (w + 2))
    srows, ssrows = None, None
    for p in range(nb // 2):
        zs = [z_ref[p, m].astype(f32) for m in range(cmid)]
        acts = [jnp.maximum(z * a1_ref[m:m + 1, :] + b1_ref[m:m + 1, :], 0.0)
                * mask for m, z in enumerate(zs)]
        pws = _pointwise(acts, pwb_ref, nout)
        sr = []
        sq = []
        for pw in pws:
            sr.append(jnp.sum(pw, axis=0, keepdims=True))
            sq.append(jnp.sum(pw * pw, axis=0, keepdims=True))
        srows = sr if srows is None else [a + b for a, b in zip(srows, sr)]
        ssrows = sq if ssrows is None else [a + b for a, b in zip(ssrows, sq)]
    s_ref[0] += jnp.concatenate(srows, axis=0)
    ss_ref[0] += jnp.concatenate(ssrows, axis=0)


def _final_kernel(z_ref, a1_ref, b1_ref, pwb_ref, a2_ref, b2_ref,
                  out_ref, *, nb, cmid, nout, h, w):
    mask = _interior_mask(h + 2, 2 * (w + 2))
    wp = w + 2
    for p in range(nb // 2):
        zs = [z_ref[p, m].astype(f32) for m in range(cmid)]
        acts = [jnp.maximum(z * a1_ref[m:m + 1, :] + b1_ref[m:m + 1, :], 0.0)
                * mask for m, z in enumerate(zs)]
        pws = _pointwise(acts, pwb_ref, nout)
        for o, pw in enumerate(pws):
            val = jnp.maximum(pw * a2_ref[o:o + 1, :] + b2_ref[o:o + 1, :],
                              0.0)
            out_ref[2 * p, o] = val[:, :wp]
            out_ref[2 * p + 1, o] = val[:, wp:]


def kernel(x, dw_w, pw_w, g1, b1, g2, b2):
    N, NIN, H, W = x.shape
    CMID = dw_w.shape[0]
    NOUT = pw_w.shape[0]
    KPL = CMID // NIN
    Hp, Wp = H + 2, W + 2
    L = 2 * Wp
    NB = 8 if N % 8 == 0 else 2
    NBLK = N // NB
    NCORE = 1
    NSEQ = NBLK // NCORE

    dwb = jnp.broadcast_to(dw_w.astype(f32).reshape(CMID * 9, 1), (CMID * 9, L))
    pwm = pw_w.astype(f32)[:, :, 0, 0]
    pwb = jnp.broadcast_to(pwm.reshape(NOUT * CMID, 1), (NOUT * CMID, L))

    x_spec = pl.BlockSpec((NB, NIN, H, W),
                          lambda k, n: (k * NSEQ + n, 0, 0, 0))

    def cspec(shape):
        nd = len(shape)
        return pl.BlockSpec(shape, lambda k, n, nd=nd: (0,) * nd)

    def accspec(shape):
        return pl.BlockSpec((1,) + shape,
                            lambda k, n: (k,) + (0,) * len(shape))

    cp = pltpu.CompilerParams(
        dimension_semantics=("arbitrary", "arbitrary"),
        vmem_limit_bytes=48 * 1024 * 1024)

    z_spec = pl.BlockSpec((NB // 2, CMID, Hp, L),
                          lambda k, n: (k * NSEQ + n, 0, 0, 0))
    s1, ss1, zc = pl.pallas_call(
        functools.partial(_stats1_kernel, nb=NB, nin=NIN, kpl=KPL, h=H, w=W),
        out_shape=(jax.ShapeDtypeStruct((NCORE, CMID, L), f32),
                   jax.ShapeDtypeStruct((NCORE, CMID, L), f32),
                   jax.ShapeDtypeStruct((N // 2, CMID, Hp, L), jnp.bfloat16)),
        grid=(NCORE, NSEQ),
        in_specs=[x_spec, cspec((CMID * 9, L))],
        out_specs=(accspec((CMID, L)), accspec((CMID, L)), z_spec),
        compiler_params=cp,
    )(x, dwb)

    cnt1 = float(N * H * W)
    mean1 = jnp.sum(s1, axis=(0, 2)) / cnt1
    var1 = jnp.maximum(jnp.sum(ss1, axis=(0, 2)) / cnt1 - mean1 * mean1, 0.0)
    a1 = g1.astype(f32) * jax.lax.rsqrt(var1 + EPS)
    b1v = b1.astype(f32) - mean1 * a1
    a1b = jnp.broadcast_to(a1.reshape(CMID, 1), (CMID, L))
    b1b = jnp.broadcast_to(b1v.reshape(CMID, 1), (CMID, L))

    s2, ss2 = pl.pallas_call(
        functools.partial(_stats2_kernel, nb=NB, cmid=CMID,
                          nout=NOUT, h=H, w=W),
        out_shape=(jax.ShapeDtypeStruct((NCORE, NOUT, L), f32),
                   jax.ShapeDtypeStruct((NCORE, NOUT, L), f32)),
        grid=(NCORE, NSEQ),
        in_specs=[z_spec, cspec((CMID, L)),
                  cspec((CMID, L)), cspec((NOUT * CMID, L))],
        out_specs=(accspec((NOUT, L)), accspec((NOUT, L))),
        compiler_params=cp,
    )(zc, a1b, b1b, pwb)

    cnt2 = float(N * Hp * Wp)
    mean2 = jnp.sum(s2, axis=(0, 2)) / cnt2
    var2 = jnp.maximum(jnp.sum(ss2, axis=(0, 2)) / cnt2 - mean2 * mean2, 0.0)
    a2 = g2.astype(f32) * jax.lax.rsqrt(var2 + EPS)
    b2v = b2.astype(f32) - mean2 * a2
    a2b = jnp.broadcast_to(a2.reshape(NOUT, 1), (NOUT, L))
    b2b = jnp.broadcast_to(b2v.reshape(NOUT, 1), (NOUT, L))

    out = pl.pallas_call(
        functools.partial(_final_kernel, nb=NB, cmid=CMID,
                          nout=NOUT, h=H, w=W),
        out_shape=jax.ShapeDtypeStruct((N, NOUT, Hp, Wp), f32),
        grid=(NCORE, NSEQ),
        in_specs=[z_spec, cspec((CMID, L)),
                  cspec((CMID, L)), cspec((NOUT * CMID, L)),
                  cspec((NOUT, L)), cspec((NOUT, L))],
        out_specs=pl.BlockSpec((NB, NOUT, Hp, Wp),
                               lambda k, n: (k * NSEQ + n, 0, 0, 0)),
        compiler_params=cp,
    )(zc, a1b, b1b, pwb, a2b, b2b)
    return out


# speedup vs baseline: 4.5093x; 1.1125x over previous
import functools

import jax
import jax.numpy as jnp
from jax.experimental import pallas as pl
from jax.experimental.pallas import tpu as pltpu

EPS = 1e-5
f32 = jnp.float32


def _shift_rows(p, d):
    if d == 0:
        return p
    if d == 1:
        return jnp.concatenate([p[1:, :], p[:1, :]], axis=0)
    return jnp.concatenate([p[-1:, :], p[:-1, :]], axis=0)


def _shift_lanes(p, d):
    if d == 0:
        return p
    if d == 1:
        return jnp.concatenate([p[:, 1:], p[:, :1]], axis=1)
    return jnp.concatenate([p[:, -1:], p[:, :-1]], axis=1)


def _build_planes(x_ref, b0, nin, h, w):
    zc = jnp.zeros((h, 1), f32)
    zr = jnp.zeros((1, 2 * (w + 2)), f32)
    planes = []
    for c in range(nin):
        row = jnp.concatenate(
            [zc, x_ref[b0, c], zc, zc, x_ref[b0 + 1, c], zc], axis=1)
        planes.append(jnp.concatenate([zr, row, zr], axis=0))
    return planes


def _depthwise(planes, dwb_ref, kpl):
    zs = []
    for c, p in enumerate(planes):
        rows = [_shift_rows(p, dh) for dh in (-1, 0, 1)]
        accs = [None] * kpl
        for kh in range(3):
            for kw in range(3):
                tap = _shift_lanes(rows[kh], kw - 1)
                j = kh * 3 + kw
                for k in range(kpl):
                    m = c * kpl + k
                    t = tap * dwb_ref[m * 9 + j:m * 9 + j + 1, :]
                    accs[k] = t if accs[k] is None else accs[k] + t
        zs.extend(accs)
    return zs


def _interior_mask(hp, l):
    wp = l // 2
    r = jax.lax.broadcasted_iota(jnp.int32, (hp, l), 0)
    q = jax.lax.broadcasted_iota(jnp.int32, (hp, l), 1)
    q = jax.lax.rem(q, wp)
    ok = (r >= 1) & (r <= hp - 2) & (q >= 1) & (q <= wp - 2)
    return jnp.where(ok, 1.0, 0.0).astype(f32)


def _pointwise(acts, pwb_ref, nout):
    cmid = len(acts)
    outs = [None] * nout
    for m in range(cmid):
        for o in range(nout):
            t = acts[m] * pwb_ref[o * cmid + m:o * cmid + m + 1, :]
            outs[o] = t if outs[o] is None else outs[o] + t
    return outs


def _stats1_kernel(x_ref, dwb_ref, s_ref, ss_ref, z_ref,
                   *, nb, nin, kpl, h, w):
    @pl.when(pl.program_id(1) == 0)
    def _():
        s_ref[...] = jnp.zeros_like(s_ref)
        ss_ref[...] = jnp.zeros_like(ss_ref)

    mask = _interior_mask(h + 2, 2 * (w + 2))
    srows, ssrows = None, None
    for p in range(nb // 2):
        zs = _depthwise(_build_planes(x_ref, 2 * p, nin, h, w), dwb_ref, kpl)
        sr = []
        sq = []
        for m, z in enumerate(zs):
            z_ref[p, m] = z.astype(jnp.bfloat16)
            zm = z * mask
            sr.append(jnp.sum(zm, axis=0, keepdims=True))
            sq.append(jnp.sum(zm * zm, axis=0, keepdims=True))
        srows = sr if srows is None else [a + b for a, b in zip(srows, sr)]
        ssrows = sq if ssrows is None else [a + b for a, b in zip(ssrows, sq)]
    s_ref[0] += jnp.concatenate(srows, axis=0)
    ss_ref[0] += jnp.concatenate(ssrows, axis=0)


def _stats2_kernel(z_ref, a1_ref, b1_ref, pwb_ref, s_ref, ss_ref,
                   *, nb, cmid, nout, h, w):
    @pl.when(pl.program_id(1) == 0)
    def _():
        s_ref[...] = jnp.zeros_like(s_ref)
        ss_ref[...] = jnp.zeros_like(ss_ref)

    mask = _interior_mask(h + 2, 2 * (w + 2))
    srows, ssrows = None, None
    for p in range(nb // 2):
        zs = [z_ref[p, m].astype(f32) for m in range(cmid)]
        acts = [jnp.maximum(z * a1_ref[m:m + 1, :] + b1_ref[m:m + 1, :], 0.0)
                * mask for m, z in enumerate(zs)]
        pws = _pointwise(acts, pwb_ref, nout)
        sr = []
        sq = []
        for pw in pws:
            sr.append(jnp.sum(pw, axis=0, keepdims=True))
            sq.append(jnp.sum(pw * pw, axis=0, keepdims=True))
        srows = sr if srows is None else [a + b for a, b in zip(srows, sr)]
        ssrows = sq if ssrows is None else [a + b for a, b in zip(ssrows, sq)]
    s_ref[0] += jnp.concatenate(srows, axis=0)
    ss_ref[0] += jnp.concatenate(ssrows, axis=0)


def _final_kernel(z_ref, a1_ref, b1_ref, pwb_ref, a2_ref, b2_ref,
                  out_ref, *, nb, cmid, nout, h, w):
    mask = _interior_mask(h + 2, 2 * (w + 2))
    wp = w + 2
    for p in range(nb // 2):
        zs = [z_ref[p, m].astype(f32) for m in range(cmid)]
        acts = [jnp.maximum(z * a1_ref[m:m + 1, :] + b1_ref[m:m + 1, :], 0.0)
                * mask for m, z in enumerate(zs)]
        pws = _pointwise(acts, pwb_ref, nout)
        for o, pw in enumerate(pws):
            val = jnp.maximum(pw * a2_ref[o:o + 1, :] + b2_ref[o:o + 1, :],
                              0.0)
            out_ref[2 * p, o] = val[:, :wp]
            out_ref[2 * p + 1, o] = val[:, wp:]


def kernel(x, dw_w, pw_w, g1, b1, g2, b2):
    N, NIN, H, W = x.shape
    CMID = dw_w.shape[0]
    NOUT = pw_w.shape[0]
    KPL = CMID // NIN
    Hp, Wp = H + 2, W + 2
    L = 2 * Wp
    NB = 32 if N % 32 == 0 else (8 if N % 8 == 0 else 2)
    NBLK = N // NB
    NCORE = 1
    NSEQ = NBLK // NCORE

    dwb = jnp.broadcast_to(dw_w.astype(f32).reshape(CMID * 9, 1), (CMID * 9, L))
    pwm = pw_w.astype(f32)[:, :, 0, 0]
    pwb = jnp.broadcast_to(pwm.reshape(NOUT * CMID, 1), (NOUT * CMID, L))

    x_spec = pl.BlockSpec((NB, NIN, H, W),
                          lambda k, n: (k * NSEQ + n, 0, 0, 0))

    def cspec(shape):
        nd = len(shape)
        return pl.BlockSpec(shape, lambda k, n, nd=nd: (0,) * nd)

    def accspec(shape):
        return pl.BlockSpec((1,) + shape,
                            lambda k, n: (k,) + (0,) * len(shape))

    cp = pltpu.CompilerParams(
        dimension_semantics=("arbitrary", "arbitrary"),
        vmem_limit_bytes=48 * 1024 * 1024)

    z_spec = pl.BlockSpec((NB // 2, CMID, Hp, L),
                          lambda k, n: (k * NSEQ + n, 0, 0, 0))
    s1, ss1, zc = pl.pallas_call(
        functools.partial(_stats1_kernel, nb=NB, nin=NIN, kpl=KPL, h=H, w=W),
        out_shape=(jax.ShapeDtypeStruct((NCORE, CMID, L), f32),
                   jax.ShapeDtypeStruct((NCORE, CMID, L), f32),
                   jax.ShapeDtypeStruct((N // 2, CMID, Hp, L), jnp.bfloat16)),
        grid=(NCORE, NSEQ),
        in_specs=[x_spec, cspec((CMID * 9, L))],
        out_specs=(accspec((CMID, L)), accspec((CMID, L)), z_spec),
        compiler_params=cp,
    )(x, dwb)

    cnt1 = float(N * H * W)
    mean1 = jnp.sum(s1, axis=(0, 2)) / cnt1
    var1 = jnp.maximum(jnp.sum(ss1, axis=(0, 2)) / cnt1 - mean1 * mean1, 0.0)
    a1 = g1.astype(f32) * jax.lax.rsqrt(var1 + EPS)
    b1v = b1.astype(f32) - mean1 * a1
    a1b = jnp.broadcast_to(a1.reshape(CMID, 1), (CMID, L))
    b1b = jnp.broadcast_to(b1v.reshape(CMID, 1), (CMID, L))

    s2, ss2 = pl.pallas_call(
        functools.partial(_stats2_kernel, nb=NB, cmid=CMID,
                          nout=NOUT, h=H, w=W),
        out_shape=(jax.ShapeDtypeStruct((NCORE, NOUT, L), f32),
                   jax.ShapeDtypeStruct((NCORE, NOUT, L), f32)),
        grid=(NCORE, NSEQ),
        in_specs=[z_spec, cspec((CMID, L)),
                  cspec((CMID, L)), cspec((NOUT * CMID, L))],
        out_specs=(accspec((NOUT, L)), accspec((NOUT, L))),
        compiler_params=cp,
    )(zc, a1b, b1b, pwb)

    cnt2 = float(N * Hp * Wp)
    mean2 = jnp.sum(s2, axis=(0, 2)) / cnt2
    var2 = jnp.maximum(jnp.sum(ss2, axis=(0, 2)) / cnt2 - mean2 * mean2, 0.0)
    a2 = g2.astype(f32) * jax.lax.rsqrt(var2 + EPS)
    b2v = b2.astype(f32) - mean2 * a2
    a2b = jnp.broadcast_to(a2.reshape(NOUT, 1), (NOUT, L))
    b2b = jnp.broadcast_to(b2v.reshape(NOUT, 1), (NOUT, L))

    out = pl.pallas_call(
        functools.partial(_final_kernel, nb=NB, cmid=CMID,
                          nout=NOUT, h=H, w=W),
        out_shape=jax.ShapeDtypeStruct((N, NOUT, Hp, Wp), f32),
        grid=(NCORE, NSEQ),
        in_specs=[z_spec, cspec((CMID, L)),
                  cspec((CMID, L)), cspec((NOUT * CMID, L)),
                  cspec((NOUT, L)), cspec((NOUT, L))],
        out_specs=pl.BlockSpec((NB, NOUT, Hp, Wp),
                               lambda k, n: (k * NSEQ + n, 0, 0, 0)),
        compiler_params=cp,
    )(zc, a1b, b1b, pwb, a2b, b2b)
    return out
